# baseline (speedup 1.0000x reference)
import jax
import jax.numpy as jnp
from jax.experimental import pallas as pl

B, C, H, W = 64, 768, 24, 24
HW = H * W
E = 8
ROWS = B * C // 2
RW = 2 * HW
RB = 2048
NSTEPS = ROWS // RB
NEG_INF = float("-inf")


def _pool_kernel(x_ref, out_ref):
    blk = x_ref[...]
    g = [blk[:, 128 * k:128 * (k + 1)] for k in range(9)]
    lane = jax.lax.broadcasted_iota(jnp.int32, (RB, 128), 1)
    in_a = lane < 64
    m_a = jnp.maximum(jnp.maximum(g[0], g[1]), jnp.maximum(g[2], g[3]))
    m_b = jnp.maximum(jnp.maximum(g[5], g[6]), jnp.maximum(g[7], g[8]))
    m_a = jnp.maximum(m_a, jnp.where(in_a, g[4], NEG_INF))
    m_b = jnp.maximum(m_b, jnp.where(in_a, NEG_INF, g[4]))
    s_a = (g[0] + g[1]) + (g[2] + g[3])
    s_b = (g[5] + g[6]) + (g[7] + g[8])
    s_a = s_a + jnp.where(in_a, g[4], 0.0)
    s_b = s_b + jnp.where(in_a, 0.0, g[4])
    pa = (jnp.max(m_a, axis=1, keepdims=True)
          + jnp.sum(s_a, axis=1, keepdims=True) * (1.0 / HW))
    pb = (jnp.max(m_b, axis=1, keepdims=True)
          + jnp.sum(s_b, axis=1, keepdims=True) * (1.0 / HW))
    out_ref[...] = jnp.concatenate([pa, pb], axis=1)


def _route_kernel(p_ref, w0_ref, b0_ref, w1_ref, b1_ref, out_ref):
    pooled = p_ref[...]
    h = jnp.dot(pooled, w0_ref[...],
                preferred_element_type=jnp.float32) + b0_ref[...]
    h = jnp.where(h >= 0.0, h, 0.2 * h)
    z = jnp.dot(pooled, w1_ref[...],
                preferred_element_type=jnp.float32) + b1_ref[...]
    noise = jnp.maximum(z, 0.0) + jnp.log1p(jnp.exp(-jnp.abs(z)))
    nmean = jnp.mean(noise, axis=1, keepdims=True)
    var = jnp.sum((noise - nmean) ** 2, axis=1, keepdims=True) / (E - 1)
    norm_noise = (noise - nmean) * jax.lax.rsqrt(var)
    scores = h + norm_noise
    ii = jax.lax.broadcasted_iota(jnp.int32, (B, E), 1)
    m1 = jnp.max(scores, axis=1, keepdims=True)
    i1 = jnp.min(jnp.where(scores == m1, ii, E), axis=1, keepdims=True)
    oh1 = ii == i1
    s2 = jnp.where(oh1, NEG_INF, scores)
    m2 = jnp.max(s2, axis=1, keepdims=True)
    i2 = jnp.min(jnp.where(s2 == m2, ii, E), axis=1, keepdims=True)
    mask = oh1 | (ii == i2)
    hm = jnp.where(mask, h, NEG_INF)
    mx = jnp.max(hm, axis=1, keepdims=True)
    e = jnp.where(mask, jnp.exp(h - mx), 0.0)
    out_ref[...] = e / jnp.sum(e, axis=1, keepdims=True)


@jax.jit
def kernel(x, W0, b0, W1, b1):
    xr = x.reshape(ROWS, RW)
    pooled2 = pl.pallas_call(
        _pool_kernel,
        grid=(NSTEPS,),
        in_specs=[pl.BlockSpec((RB, RW), lambda j: (j, 0))],
        out_specs=pl.BlockSpec((RB, 2), lambda j: (j, 0)),
        out_shape=jax.ShapeDtypeStruct((ROWS, 2), jnp.float32),
    )(xr)
    pooled = pooled2.reshape(B, C)
    return pl.pallas_call(
        _route_kernel,
        in_specs=[
            pl.BlockSpec((B, C), lambda: (0, 0)),
            pl.BlockSpec((C, E), lambda: (0, 0)),
            pl.BlockSpec((1, E), lambda: (0, 0)),
            pl.BlockSpec((C, E), lambda: (0, 0)),
            pl.BlockSpec((1, E), lambda: (0, 0)),
        ],
        out_specs=pl.BlockSpec((B, E), lambda: (0, 0)),
        out_shape=jax.ShapeDtypeStruct((B, E), jnp.float32),
    )(pooled, W0.T, b0.reshape(1, E), W1.T, b1.reshape(1, E))


# speedup vs baseline: 16.3366x; 16.3366x over previous
import jax
import jax.numpy as jnp
from jax.experimental import pallas as pl

B, C, H, W = 64, 768, 24, 24
HW = H * W
E = 8
BB = 8
NSTEPS = B // BB
NEG_INF = float("-inf")


def _pool_kernel(x_ref, out_ref):
    blk = x_ref[...]
    out_ref[...] = (jnp.max(blk, axis=1)
                    + jnp.sum(blk, axis=1) * (1.0 / HW))


def _route_kernel(p_ref, w0_ref, b0_ref, w1_ref, b1_ref, out_ref):
    pooled = p_ref[...]
    h = jnp.dot(pooled, w0_ref[...],
                preferred_element_type=jnp.float32) + b0_ref[...]
    h = jnp.where(h >= 0.0, h, 0.2 * h)
    z = jnp.dot(pooled, w1_ref[...],
                preferred_element_type=jnp.float32) + b1_ref[...]
    noise = jnp.maximum(z, 0.0) + jnp.log1p(jnp.exp(-jnp.abs(z)))
    nmean = jnp.mean(noise, axis=1, keepdims=True)
    var = jnp.sum((noise - nmean) ** 2, axis=1, keepdims=True) / (E - 1)
    norm_noise = (noise - nmean) * jax.lax.rsqrt(var)
    scores = h + norm_noise
    ii = jax.lax.broadcasted_iota(jnp.int32, (B, E), 1)
    m1 = jnp.max(scores, axis=1, keepdims=True)
    i1 = jnp.min(jnp.where(scores == m1, ii, E), axis=1, keepdims=True)
    oh1 = ii == i1
    s2 = jnp.where(oh1, NEG_INF, scores)
    m2 = jnp.max(s2, axis=1, keepdims=True)
    i2 = jnp.min(jnp.where(s2 == m2, ii, E), axis=1, keepdims=True)
    mask = oh1 | (ii == i2)
    hm = jnp.where(mask, h, NEG_INF)
    mx = jnp.max(hm, axis=1, keepdims=True)
    e = jnp.where(mask, jnp.exp(h - mx), 0.0)
    out_ref[...] = e / jnp.sum(e, axis=1, keepdims=True)


@jax.jit
def kernel(x, W0, b0, W1, b1):
    xt = jnp.transpose(x, (0, 2, 3, 1)).reshape(B, HW, C)
    pooled = pl.pallas_call(
        _pool_kernel,
        grid=(NSTEPS,),
        in_specs=[pl.BlockSpec((BB, HW, C), lambda j: (j, 0, 0))],
        out_specs=pl.BlockSpec((BB, C), lambda j: (j, 0)),
        out_shape=jax.ShapeDtypeStruct((B, C), jnp.float32),
    )(xt)
    return pl.pallas_call(
        _route_kernel,
        in_specs=[
            pl.BlockSpec((B, C), lambda: (0, 0)),
            pl.BlockSpec((C, E), lambda: (0, 0)),
            pl.BlockSpec((1, E), lambda: (0, 0)),
            pl.BlockSpec((C, E), lambda: (0, 0)),
            pl.BlockSpec((1, E), lambda: (0, 0)),
        ],
        out_specs=pl.BlockSpec((B, E), lambda: (0, 0)),
        out_shape=jax.ShapeDtypeStruct((B, E), jnp.float32),
    )(pooled, W0.T, b0.reshape(1, E), W1.T, b1.reshape(1, E))


# speedup vs baseline: 16.9292x; 1.0363x over previous
import jax
import jax.numpy as jnp
from jax.experimental import pallas as pl
from jax.experimental.pallas import tpu as pltpu

B, C, H, W = 64, 768, 24, 24
HW = H * W
E = 8
BB = 8
NSTEPS = B // BB
NEG_INF = float("-inf")


def _gate_kernel(x_ref, w0_ref, b0_ref, w1_ref, b1_ref, out_ref, acc):
    j = pl.program_id(0)
    blk = x_ref[...]
    acc[pl.ds(j * BB, BB), :] = (jnp.max(blk, axis=1)
                                 + jnp.sum(blk, axis=1) * (1.0 / HW))

    @pl.when(j == NSTEPS - 1)
    def _epilogue():
        pooled = acc[...]
        h = jnp.dot(pooled, w0_ref[...],
                    preferred_element_type=jnp.float32) + b0_ref[...]
        h = jnp.where(h >= 0.0, h, 0.2 * h)
        z = jnp.dot(pooled, w1_ref[...],
                    preferred_element_type=jnp.float32) + b1_ref[...]
        noise = jnp.maximum(z, 0.0) + jnp.log1p(jnp.exp(-jnp.abs(z)))
        nmean = jnp.mean(noise, axis=1, keepdims=True)
        var = jnp.sum((noise - nmean) ** 2, axis=1, keepdims=True) / (E - 1)
        norm_noise = (noise - nmean) * jax.lax.rsqrt(var)
        scores = h + norm_noise
        ii = jax.lax.broadcasted_iota(jnp.int32, (B, E), 1)
        m1 = jnp.max(scores, axis=1, keepdims=True)
        i1 = jnp.min(jnp.where(scores == m1, ii, E), axis=1, keepdims=True)
        oh1 = ii == i1
        s2 = jnp.where(oh1, NEG_INF, scores)
        m2 = jnp.max(s2, axis=1, keepdims=True)
        i2 = jnp.min(jnp.where(s2 == m2, ii, E), axis=1, keepdims=True)
        mask = oh1 | (ii == i2)
        hm = jnp.where(mask, h, NEG_INF)
        mx = jnp.max(hm, axis=1, keepdims=True)
        e = jnp.where(mask, jnp.exp(h - mx), 0.0)
        gate = e / jnp.sum(e, axis=1, keepdims=True)
        out_ref[...] = gate.T


@jax.jit
def kernel(x, W0, b0, W1, b1):
    xt = jnp.transpose(x, (0, 2, 3, 1)).reshape(B, HW, C)
    gate_t = pl.pallas_call(
        _gate_kernel,
        grid=(NSTEPS,),
        in_specs=[
            pl.BlockSpec((BB, HW, C), lambda j: (j, 0, 0)),
            pl.BlockSpec((C, E), lambda j: (0, 0)),
            pl.BlockSpec((1, E), lambda j: (0, 0)),
            pl.BlockSpec((C, E), lambda j: (0, 0)),
            pl.BlockSpec((1, E), lambda j: (0, 0)),
        ],
        out_specs=pl.BlockSpec((E, B), lambda j: (0, 0)),
        out_shape=jax.ShapeDtypeStruct((E, B), jnp.float32),
        scratch_shapes=[pltpu.VMEM((B, C), jnp.float32)],
    )(xt, W0.T, b0.reshape(1, E), W1.T, b1.reshape(1, E))
    return gate_t.T
